# Initial kernel scaffold; baseline (speedup 1.0000x reference)
import jax
import jax.numpy as jnp
from jax.experimental import pallas as pl


def kernel(xyz0, xyz1, xyz2, x0, x1, x2):
    raise NotImplementedError("write your pallas kernel here")


# speedup vs baseline: 24.2976x; 24.2976x over previous
import functools

import jax
import jax.numpy as jnp
from jax.experimental import pallas as pl


def _prop_body(qT_ref, s_ref, xskip_ref, feats_ref, out_ref, *, M, C1, K):
    qT = qT_ref[...]
    s = s_ref[...]
    qn = jnp.sum(qT * qT, axis=0, keepdims=True)
    sn = jnp.sum(s * s, axis=1, keepdims=True)
    cross = jax.lax.dot_general(s, qT, (((1,), (0,)), ((), ())),
                                preferred_element_type=jnp.float32)
    d = sn + qn - 2.0 * cross
    bits = jax.lax.bitcast_convert_type(d, jnp.int32)
    key = bits ^ ((bits >> 31) & jnp.int32(0x7FFFFFFF))
    wsum = jnp.zeros_like(qn)
    m = jnp.full_like(qn, -0x80000000, dtype=jnp.int32)
    for _ in range(K):
        m = jnp.min(jnp.where(key > m, key, jnp.int32(0x7FFFFFFF)),
                    axis=0, keepdims=True)
        bits_m = m ^ ((m >> 31) & jnp.int32(0x7FFFFFFF))
        dm = jax.lax.bitcast_convert_type(bits_m, jnp.float32)
        wsum = wsum + 1.0 / (dm + 1e-8)
    W = jnp.where(key <= m, 1.0 / (d + 1e-8), 0.0)
    interp = jnp.dot(feats_ref[...], W, precision=jax.lax.Precision.HIGHEST,
                     preferred_element_type=jnp.float32)
    out_ref[0:C1, :] = xskip_ref[...]
    out_ref[C1:, :] = interp / wsum


def _propagate(q_xyz, s_xyz, x_skip, feats, qb):
    B, N, _ = q_xyz.shape
    M = s_xyz.shape[1]
    C1 = x_skip.shape[1]
    C = feats.shape[1]
    qT = jnp.transpose(q_xyz, (0, 2, 1))
    body = functools.partial(_prop_body, M=M, C1=C1, K=6)
    return pl.pallas_call(
        body,
        grid=(B, N // qb),
        in_specs=[
            pl.BlockSpec((None, 3, qb), lambda b, n: (b, 0, n)),
            pl.BlockSpec((None, M, 3), lambda b, n: (b, 0, 0)),
            pl.BlockSpec((None, C1, qb), lambda b, n: (b, 0, n)),
            pl.BlockSpec((None, C, M), lambda b, n: (b, 0, 0)),
        ],
        out_specs=pl.BlockSpec((None, C1 + C, qb), lambda b, n: (b, 0, n)),
        out_shape=jax.ShapeDtypeStruct((B, C1 + C, N), jnp.float32),
    )(qT, s_xyz, x_skip, feats)


def kernel(xyz0, xyz1, xyz2, x0, x1, x2):
    g1 = _propagate(xyz1, xyz2, x1, x2, qb=512)
    g0 = _propagate(xyz0, xyz1, x0, g1, qb=512)
    return g0


# speedup vs baseline: 26.3578x; 1.0848x over previous
import functools

import jax
import jax.numpy as jnp
from jax.experimental import pallas as pl


def _prop_body(qT_ref, s_ref, xskip_ref, feats_ref, out_ref, *, M, C1, K):
    qT = qT_ref[...]
    s = s_ref[...]
    qn = jnp.sum(qT * qT, axis=0, keepdims=True)
    sn = jnp.sum(s * s, axis=1, keepdims=True)
    cross = jax.lax.dot_general(s, qT, (((1,), (0,)), ((), ())),
                                preferred_element_type=jnp.float32)
    d = sn + qn - 2.0 * cross
    bits = jax.lax.bitcast_convert_type(d, jnp.int32)
    key = bits ^ ((bits >> 31) & jnp.int32(0x7FFFFFFF))
    wsum = jnp.zeros_like(qn)
    m = jnp.full_like(qn, -0x80000000, dtype=jnp.int32)
    for _ in range(K):
        m = jnp.min(jnp.where(key > m, key, jnp.int32(0x7FFFFFFF)),
                    axis=0, keepdims=True)
        bits_m = m ^ ((m >> 31) & jnp.int32(0x7FFFFFFF))
        dm = jax.lax.bitcast_convert_type(bits_m, jnp.float32)
        wsum = wsum + 1.0 / (dm + 1e-8)
    W = jnp.where(key <= m, 1.0 / (d + 1e-8), 0.0)
    interp = jnp.dot(feats_ref[...], W, precision=jax.lax.Precision.HIGHEST,
                     preferred_element_type=jnp.float32)
    out_ref[0:C1, :] = xskip_ref[...]
    out_ref[C1:, :] = interp / wsum


def _propagate(q_xyz, s_xyz, x_skip, feats, qb):
    B, N, _ = q_xyz.shape
    M = s_xyz.shape[1]
    C1 = x_skip.shape[1]
    C = feats.shape[1]
    qT = jnp.transpose(q_xyz, (0, 2, 1))
    body = functools.partial(_prop_body, M=M, C1=C1, K=6)
    return pl.pallas_call(
        body,
        grid=(B, N // qb),
        in_specs=[
            pl.BlockSpec((None, 3, qb), lambda b, n: (b, 0, n)),
            pl.BlockSpec((None, M, 3), lambda b, n: (b, 0, 0)),
            pl.BlockSpec((None, C1, qb), lambda b, n: (b, 0, n)),
            pl.BlockSpec((None, C, M), lambda b, n: (b, 0, 0)),
        ],
        out_specs=pl.BlockSpec((None, C1 + C, qb), lambda b, n: (b, 0, n)),
        out_shape=jax.ShapeDtypeStruct((B, C1 + C, N), jnp.float32),
    )(qT, s_xyz, x_skip, feats)


def kernel(xyz0, xyz1, xyz2, x0, x1, x2):
    g1 = _propagate(xyz1, xyz2, x1, x2, qb=1024)
    g0 = _propagate(xyz0, xyz1, x0, g1, qb=1024)
    return g0


# speedup vs baseline: 27.2223x; 1.0328x over previous
import functools

import jax
import jax.numpy as jnp
from jax.experimental import pallas as pl


def _prop_body(qT_ref, s_ref, xskip_ref, feats_ref, out_ref, *, M, C1, K):
    qT = qT_ref[...]
    s = s_ref[...]
    qn = jnp.sum(qT * qT, axis=0, keepdims=True)
    sn = jnp.sum(s * s, axis=1, keepdims=True)
    cross = jax.lax.dot_general(s, qT, (((1,), (0,)), ((), ())),
                                preferred_element_type=jnp.float32)
    d = sn + qn - 2.0 * cross
    bits = jax.lax.bitcast_convert_type(d, jnp.int32)
    key = bits ^ ((bits >> 31) & jnp.int32(0x7FFFFFFF))
    wsum = jnp.zeros_like(qn)
    m = jnp.full_like(qn, -0x80000000, dtype=jnp.int32)
    for _ in range(K):
        m = jnp.min(jnp.where(key > m, key, jnp.int32(0x7FFFFFFF)),
                    axis=0, keepdims=True)
        bits_m = m ^ ((m >> 31) & jnp.int32(0x7FFFFFFF))
        dm = jax.lax.bitcast_convert_type(bits_m, jnp.float32)
        wsum = wsum + 1.0 / (dm + 1e-8)
    W = jnp.where(key <= m, 1.0 / (d + 1e-8), 0.0)
    interp = jnp.dot(feats_ref[...], W, precision=jax.lax.Precision.HIGHEST,
                     preferred_element_type=jnp.float32)
    out_ref[0:C1, :] = xskip_ref[...]
    out_ref[C1:, :] = interp / wsum


def _propagate(q_xyz, s_xyz, x_skip, feats, qb):
    B, N, _ = q_xyz.shape
    M = s_xyz.shape[1]
    C1 = x_skip.shape[1]
    C = feats.shape[1]
    qT = jnp.transpose(q_xyz, (0, 2, 1))
    body = functools.partial(_prop_body, M=M, C1=C1, K=6)
    return pl.pallas_call(
        body,
        grid=(B, N // qb),
        in_specs=[
            pl.BlockSpec((None, 3, qb), lambda b, n: (b, 0, n)),
            pl.BlockSpec((None, M, 3), lambda b, n: (b, 0, 0)),
            pl.BlockSpec((None, C1, qb), lambda b, n: (b, 0, n)),
            pl.BlockSpec((None, C, M), lambda b, n: (b, 0, 0)),
        ],
        out_specs=pl.BlockSpec((None, C1 + C, qb), lambda b, n: (b, 0, n)),
        out_shape=jax.ShapeDtypeStruct((B, C1 + C, N), jnp.float32),
    )(qT, s_xyz, x_skip, feats)


def kernel(xyz0, xyz1, xyz2, x0, x1, x2):
    g1 = _propagate(xyz1, xyz2, x1, x2, qb=1024)
    g0 = _propagate(xyz0, xyz1, x0, g1, qb=2048)
    return g0


# speedup vs baseline: 57.8738x; 2.1260x over previous
import functools

import jax
import jax.numpy as jnp
from jax.experimental import pallas as pl


def _prop_body(qT_ref, s_ref, xskip_ref, feats_ref, out_ref, *, M, C1, K):
    qT = qT_ref[...]
    s = s_ref[...]
    qn = jnp.sum(qT * qT, axis=0, keepdims=True)
    sn = jnp.sum(s * s, axis=1, keepdims=True)
    cross = jax.lax.dot_general(s, qT, (((1,), (0,)), ((), ())),
                                preferred_element_type=jnp.float32)
    d = sn + qn - 2.0 * cross
    bits = jax.lax.bitcast_convert_type(d, jnp.int32)
    key = bits ^ ((bits >> 31) & jnp.int32(0x7FFFFFFF))
    wsum = jnp.zeros_like(qn)
    m = jnp.full_like(qn, -0x80000000, dtype=jnp.int32)
    for _ in range(K):
        m = jnp.min(jnp.where(key > m, key, jnp.int32(0x7FFFFFFF)),
                    axis=0, keepdims=True)
        bits_m = m ^ ((m >> 31) & jnp.int32(0x7FFFFFFF))
        dm = jax.lax.bitcast_convert_type(bits_m, jnp.float32)
        wsum = wsum + 1.0 / (dm + 1e-8)
    W = jnp.where(key <= m, 1.0 / (d + 1e-8), 0.0)
    interp = jnp.dot(feats_ref[...], W, preferred_element_type=jnp.float32)
    out_ref[0:C1, :] = xskip_ref[...]
    out_ref[C1:, :] = interp / wsum


def _propagate(q_xyz, s_xyz, x_skip, feats, qb):
    B, N, _ = q_xyz.shape
    M = s_xyz.shape[1]
    C1 = x_skip.shape[1]
    C = feats.shape[1]
    qT = jnp.transpose(q_xyz, (0, 2, 1))
    body = functools.partial(_prop_body, M=M, C1=C1, K=6)
    return pl.pallas_call(
        body,
        grid=(B, N // qb),
        in_specs=[
            pl.BlockSpec((None, 3, qb), lambda b, n: (b, 0, n)),
            pl.BlockSpec((None, M, 3), lambda b, n: (b, 0, 0)),
            pl.BlockSpec((None, C1, qb), lambda b, n: (b, 0, n)),
            pl.BlockSpec((None, C, M), lambda b, n: (b, 0, 0)),
        ],
        out_specs=pl.BlockSpec((None, C1 + C, qb), lambda b, n: (b, 0, n)),
        out_shape=jax.ShapeDtypeStruct((B, C1 + C, N), jnp.float32),
    )(qT, s_xyz, x_skip, feats)


def kernel(xyz0, xyz1, xyz2, x0, x1, x2):
    g1 = _propagate(xyz1, xyz2, x1, x2, qb=1024)
    g0 = _propagate(xyz0, xyz1, x0, g1, qb=2048)
    return g0


# speedup vs baseline: 58.3025x; 1.0074x over previous
import functools

import jax
import jax.numpy as jnp
from jax.experimental import pallas as pl


def _prop_body(qT_ref, s_ref, xskip_ref, feats_ref, out_ref, *, M, C1, K):
    qT = qT_ref[...]
    s = s_ref[...]
    qn = jnp.sum(qT * qT, axis=0, keepdims=True)
    sn = jnp.sum(s * s, axis=1, keepdims=True)
    cross = jax.lax.dot_general(s, qT, (((1,), (0,)), ((), ())),
                                preferred_element_type=jnp.float32)
    d = sn + qn - 2.0 * cross
    bits = jax.lax.bitcast_convert_type(d, jnp.int32)
    key = bits ^ ((bits >> 31) & jnp.int32(0x7FFFFFFF))
    wsum = jnp.zeros_like(qn)
    m = jnp.full_like(qn, -0x80000000, dtype=jnp.int32)
    for _ in range(K):
        m = jnp.min(jnp.where(key > m, key, jnp.int32(0x7FFFFFFF)),
                    axis=0, keepdims=True)
        bits_m = m ^ ((m >> 31) & jnp.int32(0x7FFFFFFF))
        dm = jax.lax.bitcast_convert_type(bits_m, jnp.float32)
        wsum = wsum + 1.0 / (dm + 1e-8)
    W = jnp.where(key <= m, 1.0 / (d + 1e-8), 0.0)
    interp = jnp.dot(feats_ref[...], W, preferred_element_type=jnp.float32)
    odt = out_ref.dtype
    out_ref[0:C1, :] = xskip_ref[...].astype(odt)
    out_ref[C1:, :] = (interp / wsum).astype(odt)


def _propagate(q_xyz, s_xyz, x_skip, feats, qb, out_dtype=jnp.float32):
    B, N, _ = q_xyz.shape
    M = s_xyz.shape[1]
    C1 = x_skip.shape[1]
    C = feats.shape[1]
    qT = jnp.transpose(q_xyz, (0, 2, 1))
    body = functools.partial(_prop_body, M=M, C1=C1, K=6)
    return pl.pallas_call(
        body,
        grid=(B, N // qb),
        in_specs=[
            pl.BlockSpec((None, 3, qb), lambda b, n: (b, 0, n)),
            pl.BlockSpec((None, M, 3), lambda b, n: (b, 0, 0)),
            pl.BlockSpec((None, C1, qb), lambda b, n: (b, 0, n)),
            pl.BlockSpec((None, C, M), lambda b, n: (b, 0, 0)),
        ],
        out_specs=pl.BlockSpec((None, C1 + C, qb), lambda b, n: (b, 0, n)),
        out_shape=jax.ShapeDtypeStruct((B, C1 + C, N), out_dtype),
    )(qT, s_xyz, x_skip, feats)


def kernel(xyz0, xyz1, xyz2, x0, x1, x2):
    g1 = _propagate(xyz1, xyz2, x1, x2, qb=1024, out_dtype=jnp.bfloat16)
    g0 = _propagate(xyz0, xyz1, x0, g1, qb=2048)
    return g0
